# baseline (speedup 1.0000x reference)
import functools

import jax
import jax.numpy as jnp
from jax import lax
from jax.experimental import pallas as pl
from jax.experimental.pallas import tpu as pltpu
from jax.experimental.pallas import tpu_sc as plsc

_NC = 2
_NS = 16
_NW = _NC * _NS
_LANES = 16
_CH = 128


def _proj_body(x_ref, w_ref, ps_ref, pr_ref):
    ps_ref[...] = jnp.dot(x_ref[0], w_ref[0],
                          preferred_element_type=jnp.float32)
    pr_ref[...] = jnp.dot(x_ref[1], w_ref[1],
                          preferred_element_type=jnp.float32)


def _node_proj(x, w_ab, *, block_n):
    _, n, h = x.shape
    grid = (n // block_n,)
    return pl.pallas_call(
        _proj_body,
        grid=grid,
        in_specs=[
            pl.BlockSpec((2, block_n, h), lambda i: (0, i, 0)),
            pl.BlockSpec((2, h, h), lambda i: (0, 0, 0)),
        ],
        out_specs=[
            pl.BlockSpec((block_n, h), lambda i: (i, 0)),
            pl.BlockSpec((block_n, h), lambda i: (i, 0)),
        ],
        out_shape=[
            jax.ShapeDtypeStruct((n, h), jnp.float32),
            jax.ShapeDtypeStruct((n, h), jnp.float32),
        ],
    )(x, w_ab)


_GCH = 64


def _gather_add(ps, pr, idx_s, idx_r, *, interpret=False):
    n, h = ps.shape
    e = idx_s.shape[0]
    nch = e // _GCH
    mesh = plsc.VectorSubcoreMesh(core_axis_name="c", subcore_axis_name="s",
                                  num_cores=_NC, num_subcores=_NS)

    @functools.partial(
        pl.kernel,
        out_type=jax.ShapeDtypeStruct((e, h), jnp.float32),
        mesh=mesh,
        compiler_params=pltpu.CompilerParams(needs_layout_passes=False),
        scratch_types=[
            [pltpu.VMEM((_GCH,), jnp.int32)] * 2,
            [pltpu.VMEM((_GCH,), jnp.int32)] * 2,
            [pltpu.VMEM((_GCH, h), jnp.float32)] * 2,
            [pltpu.VMEM((_GCH, h), jnp.float32)] * 2,
            [pltpu.SemaphoreType.DMA] * 2,
            [pltpu.SemaphoreType.DMA] * 2,
        ],
        interpret=interpret,
    )
    def k(ps_hbm, pr_hbm, is_hbm, ir_hbm, pre_hbm,
          idx_sv, idx_rv, ra, rb, sem_a, sem_b):
        w = lax.axis_index("s") * _NC + lax.axis_index("c")
        trips = nch // _NW + jnp.where(w < nch % _NW, 1, 0)
        nj = h // _LANES

        def issue(kk, p):
            base = pl.multiple_of((w + _NW * kk) * _GCH, _GCH)
            pltpu.sync_copy(is_hbm.at[pl.ds(base, _GCH)], idx_sv[p])
            pltpu.sync_copy(ir_hbm.at[pl.ds(base, _GCH)], idx_rv[p])
            pltpu.async_copy(ps_hbm.at[idx_sv[p]], ra[p], sem_a[p])
            pltpu.async_copy(pr_hbm.at[idx_rv[p]], rb[p], sem_b[p])

        def process(kk, p):
            base = pl.multiple_of((w + _NW * kk) * _GCH, _GCH)
            pltpu.make_async_copy(ps_hbm.at[idx_sv[p]], ra[p],
                                  sem_a[p]).wait()
            pltpu.make_async_copy(pr_hbm.at[idx_rv[p]], rb[p],
                                  sem_b[p]).wait()

            def add_row(row, _):
                sls = [pl.ds(j * _LANES, _LANES) for j in range(nj)]
                avs = [ra[p][row, sl] for sl in sls]
                bvs = [rb[p][row, sl] for sl in sls]
                for j in range(nj):
                    ra[p][row, sls[j]] = avs[j] + bvs[j]
                return 0

            lax.fori_loop(0, _GCH, add_row, 0)
            pltpu.sync_copy(ra[p], pre_hbm.at[pl.ds(base, _GCH)])

        @pl.when(trips > 0)
        def _():
            issue(0, 0)

        def body(kp, _):
            for p in range(2):
                kcur = 2 * kp + p

                @pl.when(kcur < trips)
                def _(p=p, kcur=kcur):
                    @pl.when(kcur + 1 < trips)
                    def _():
                        issue(kcur + 1, 1 - p)
                    process(kcur, p)
            return 0

        lax.fori_loop(0, (trips + 1) // 2, body, 0)

    return k(ps, pr, idx_s, idx_r)


def _mlp_body(pre_ref, ea_ref, w1c_ref, b1_ref, w2_ref, b2_ref, w3_ref,
              b3_ref, out_ref):
    pre1 = (pre_ref[...]
            + jnp.dot(ea_ref[...], w1c_ref[...],
                      preferred_element_type=jnp.float32)
            + b1_ref[...])
    hmid = pre1 * jax.nn.sigmoid(pre1)
    m2 = jnp.dot(hmid.astype(jnp.bfloat16), w2_ref[...],
                 preferred_element_type=jnp.float32) + b2_ref[...]
    msg = m2 * jax.nn.sigmoid(m2)
    gate = jnp.sum(msg * w3_ref[...], axis=1, keepdims=True) + b3_ref[...]
    out_ref[...] = msg * jax.nn.sigmoid(gate)


def _edge_mlp(pre, ea, w1c, b1, w2, b2, w3t, b3, *, block_e):
    e, h = pre.shape
    ni = ea.shape[1]
    grid = (e // block_e,)
    return pl.pallas_call(
        _mlp_body,
        grid=grid,
        in_specs=[
            pl.BlockSpec((block_e, h), lambda i: (i, 0)),
            pl.BlockSpec((block_e, ni), lambda i: (i, 0)),
            pl.BlockSpec((ni, h), lambda i: (0, 0)),
            pl.BlockSpec((1, h), lambda i: (0, 0)),
            pl.BlockSpec((h, h), lambda i: (0, 0)),
            pl.BlockSpec((1, h), lambda i: (0, 0)),
            pl.BlockSpec((1, h), lambda i: (0, 0)),
            pl.BlockSpec((1, 1), lambda i: (0, 0)),
        ],
        out_specs=pl.BlockSpec((block_e, h), lambda i: (i, 0)),
        out_shape=jax.ShapeDtypeStruct((e, h), jnp.float32),
    )(pre, ea, w1c, b1, w2, b2, w3t, b3)


_SCH = 640
_OWN = 320
_BUF = 160
_FB = 64


def _scatter_add(wmsg, idx_r, n, *, interpret=False):
    e, h = wmsg.shape
    own = (n + _NW - 1) // _NW
    mesh = plsc.VectorSubcoreMesh(core_axis_name="c", subcore_axis_name="s",
                                  num_cores=_NC, num_subcores=_NS)

    @functools.partial(
        pl.kernel,
        out_type=jax.ShapeDtypeStruct((_NW, _OWN, h), jnp.float32),
        mesh=mesh,
        compiler_params=pltpu.CompilerParams(needs_layout_passes=False),
        scratch_types=[
            pltpu.VMEM((_SCH,), jnp.int32),
            pltpu.VMEM((_BUF,), jnp.int32),
            pltpu.VMEM((2 * _FB,), jnp.int32),
            pltpu.VMEM((2 * _FB,), jnp.int32),
            pltpu.VMEM((2 * _FB, h), jnp.float32),
            pltpu.VMEM((_OWN, h), jnp.float32),
            pltpu.SemaphoreType.DMA,
        ],
        interpret=interpret,
    )
    def k(w_hbm, ir_hbm, z_hbm, out_hbm, idxb, pb, eidb, ob, rows, acc, sem):
        c = lax.axis_index("c")
        s = lax.axis_index("s")
        wid = s * _NC + c
        lo = wid * own
        iota = lax.iota(jnp.int32, _LANES)
        zero16 = jnp.zeros((_LANES,), jnp.int32)

        for r0 in range(0, _OWN, 2 * _FB):
            step = min(2 * _FB, _OWN - r0)
            pltpu.sync_copy(z_hbm.at[pl.ds(0, step)],
                            acc.at[pl.ds(r0, step)])
        for r0 in range(0, _BUF, _LANES):
            pb[pl.ds(r0, _LANES)] = zero16

        nj = h // _LANES

        def accumulate(bs, cnt):
            def grp(q, _):
                ov = ob[pl.ds(bs + q * _LANES, _LANES)]
                dv = jnp.where(q * _LANES + iota < cnt, ov, own)
                drow = [dv[ln] for ln in range(_LANES)]
                for ln in range(_LANES):
                    d = drow[ln]
                    r = bs + q * _LANES + ln
                    sls = [pl.ds(j * _LANES, _LANES) for j in range(nj)]
                    avs = [acc[d, sl] for sl in sls]
                    rvs = [rows[r, sl] for sl in sls]
                    for j in range(nj):
                        acc[d, sls[j]] = avs[j] + rvs[j]
                return 0
            lax.fori_loop(0, _FB // _LANES, grp, 0)

        def wait_and_accumulate(bs):
            pltpu.make_async_copy(
                w_hbm.at[eidb.at[pl.ds(bs, _FB)]],
                rows.at[pl.ds(bs, _FB)], sem).wait()
            accumulate(bs, _FB)

        def issue(bs):
            for q in range(_FB // _LANES):
                pv = pb[pl.ds(q * _LANES, _LANES)]
                eidb[pl.ds(bs + q * _LANES, _LANES)] = pv >> 9
                ob[pl.ds(bs + q * _LANES, _LANES)] = pv & 0x1FF
            pltpu.async_copy(w_hbm.at[eidb.at[pl.ds(bs, _FB)]],
                             rows.at[pl.ds(bs, _FB)], sem)

        def scan_chunk(ci, carry):
            wp, fc = carry
            base = ci * _SCH
            pltpu.sync_copy(ir_hbm.at[pl.ds(base, _SCH)], idxb)

            def group(g2, carry):
                wp, fc = carry
                for u in range(4):
                    g = g2 * 4 + u
                    v = idxb[pl.ds(g * _LANES, _LANES)]
                    off = v - lo
                    m = (off >= 0) & (off < own)
                    cnt = plsc.all_reduce_population_count(m)[0]

                    @pl.when(cnt > 0)
                    def _(wp=wp, off=off, m=m, g=g):
                        packed = ((base + g * _LANES + iota) << 9) \
                            | (off & 0x1FF)
                        plsc.store_compressed(pb.at[pl.ds(wp, _LANES)],
                                              packed, mask=m)

                    wp = wp + cnt

                @pl.when(wp >= _FB)
                def _():
                    bs = (fc & 1) * _FB

                    @pl.when(fc > 0)
                    def _():
                        wait_and_accumulate(_FB - bs)

                    issue(bs)
                    pvs = [pb[pl.ds(_FB + q * _LANES, _LANES)]
                           for q in range(4)]
                    for q in range(4):
                        pb[pl.ds(q * _LANES, _LANES)] = pvs[q]

                hit = wp >= _FB
                return (jnp.where(hit, wp - _FB, wp),
                        jnp.where(hit, fc + 1, fc))

            return lax.fori_loop(0, _SCH // _LANES // 4, group, (wp, fc))

        wp, fc = lax.fori_loop(0, e // _SCH, scan_chunk,
                               (jnp.int32(0), jnp.int32(0)))

        @pl.when(fc > 0)
        def _():
            wait_and_accumulate(_FB - (fc & 1) * _FB)

        bs = (fc & 1) * _FB
        issue(bs)
        pltpu.make_async_copy(w_hbm.at[eidb.at[pl.ds(bs, _FB)]],
                              rows.at[pl.ds(bs, _FB)], sem).wait()
        accumulate(bs, wp)
        pltpu.sync_copy(acc, out_hbm.at[wid])

    out = k(wmsg, idx_r, jnp.zeros((2 * _FB, h), jnp.float32))
    return out[:, :own, :].reshape(_NW * own, h)[:n]


def kernel(x, index, edge_attr, W1, b1, W2, b2, W3, b3):
    n, h = x.shape[1], x.shape[2]
    w_ab = jnp.stack([W1[:h], W1[h:2 * h]])
    ps, pr = _node_proj(x, w_ab, block_n=2000)
    pre = _gather_add(ps, pr, index[0], index[1])
    wmsg = _edge_mlp(pre, edge_attr, W1[2 * h:], b1.reshape(1, h),
                     W2.astype(jnp.bfloat16), b2.reshape(1, h),
                     W3.reshape(1, h), b3.reshape(1, 1), block_e=1600)
    return _scatter_add(wmsg, index[1], n)


# speedup vs baseline: 1.1763x; 1.1763x over previous
import functools

import jax
import jax.numpy as jnp
from jax import lax
from jax.experimental import pallas as pl
from jax.experimental.pallas import tpu as pltpu
from jax.experimental.pallas import tpu_sc as plsc

_NC = 2
_NS = 16
_NW = _NC * _NS
_LANES = 16
_CH = 128


def _proj_body(x_ref, w_ref, ps_ref, pr_ref):
    ps_ref[...] = jnp.dot(x_ref[0], w_ref[0],
                          preferred_element_type=jnp.float32)
    pr_ref[...] = jnp.dot(x_ref[1], w_ref[1],
                          preferred_element_type=jnp.float32)


def _node_proj(x, w_ab, *, block_n):
    _, n, h = x.shape
    grid = (n // block_n,)
    return pl.pallas_call(
        _proj_body,
        grid=grid,
        in_specs=[
            pl.BlockSpec((2, block_n, h), lambda i: (0, i, 0)),
            pl.BlockSpec((2, h, h), lambda i: (0, 0, 0)),
        ],
        out_specs=[
            pl.BlockSpec((block_n, h), lambda i: (i, 0)),
            pl.BlockSpec((block_n, h), lambda i: (i, 0)),
        ],
        out_shape=[
            jax.ShapeDtypeStruct((n, h), jnp.float32),
            jax.ShapeDtypeStruct((n, h), jnp.float32),
        ],
    )(x, w_ab)


_GCH = 64


def _gather_add(ps, pr, idx_s, idx_r, *, interpret=False):
    n, h = ps.shape
    e = idx_s.shape[0]
    nch = e // _GCH
    mesh = plsc.VectorSubcoreMesh(core_axis_name="c", subcore_axis_name="s",
                                  num_cores=_NC, num_subcores=_NS)

    @functools.partial(
        pl.kernel,
        out_type=jax.ShapeDtypeStruct((e, h), jnp.float32),
        mesh=mesh,
        compiler_params=pltpu.CompilerParams(needs_layout_passes=False),
        scratch_types=[
            [pltpu.VMEM((_GCH,), jnp.int32)] * 2,
            [pltpu.VMEM((_GCH,), jnp.int32)] * 2,
            [pltpu.VMEM((_GCH, h), jnp.float32)] * 2,
            [pltpu.VMEM((_GCH, h), jnp.float32)] * 2,
            [pltpu.SemaphoreType.DMA] * 2,
            [pltpu.SemaphoreType.DMA] * 2,
        ],
        interpret=interpret,
    )
    def k(ps_hbm, pr_hbm, is_hbm, ir_hbm, pre_hbm,
          idx_sv, idx_rv, ra, rb, sem_a, sem_b):
        w = lax.axis_index("s") * _NC + lax.axis_index("c")
        trips = nch // _NW + jnp.where(w < nch % _NW, 1, 0)
        nj = h // _LANES

        def issue(kk, p):
            base = pl.multiple_of((w + _NW * kk) * _GCH, _GCH)
            pltpu.sync_copy(is_hbm.at[pl.ds(base, _GCH)], idx_sv[p])
            pltpu.sync_copy(ir_hbm.at[pl.ds(base, _GCH)], idx_rv[p])
            pltpu.async_copy(ps_hbm.at[idx_sv[p]], ra[p], sem_a[p])
            pltpu.async_copy(pr_hbm.at[idx_rv[p]], rb[p], sem_b[p])

        def process(kk, p):
            base = pl.multiple_of((w + _NW * kk) * _GCH, _GCH)
            pltpu.make_async_copy(ps_hbm.at[idx_sv[p]], ra[p],
                                  sem_a[p]).wait()
            pltpu.make_async_copy(pr_hbm.at[idx_rv[p]], rb[p],
                                  sem_b[p]).wait()

            def add_row(row, _):
                sls = [pl.ds(j * _LANES, _LANES) for j in range(nj)]
                avs = [ra[p][row, sl] for sl in sls]
                bvs = [rb[p][row, sl] for sl in sls]
                for j in range(nj):
                    ra[p][row, sls[j]] = avs[j] + bvs[j]
                return 0

            lax.fori_loop(0, _GCH, add_row, 0)
            pltpu.sync_copy(ra[p], pre_hbm.at[pl.ds(base, _GCH)])

        @pl.when(trips > 0)
        def _():
            issue(0, 0)

        def body(kp, _):
            for p in range(2):
                kcur = 2 * kp + p

                @pl.when(kcur < trips)
                def _(p=p, kcur=kcur):
                    @pl.when(kcur + 1 < trips)
                    def _():
                        issue(kcur + 1, 1 - p)
                    process(kcur, p)
            return 0

        lax.fori_loop(0, (trips + 1) // 2, body, 0)

    return k(ps, pr, idx_s, idx_r)


def _mlp_body(pre_ref, ea_ref, w1c_ref, b1_ref, w2_ref, b2_ref, w3_ref,
              b3_ref, out_ref):
    pre1 = (pre_ref[...]
            + jnp.dot(ea_ref[...], w1c_ref[...],
                      preferred_element_type=jnp.float32)
            + b1_ref[...])
    hmid = pre1 * jax.nn.sigmoid(pre1)
    m2 = jnp.dot(hmid.astype(jnp.bfloat16), w2_ref[...],
                 preferred_element_type=jnp.float32) + b2_ref[...]
    msg = m2 * jax.nn.sigmoid(m2)
    gate = jnp.sum(msg * w3_ref[...], axis=1, keepdims=True) + b3_ref[...]
    out_ref[...] = msg * jax.nn.sigmoid(gate)


def _edge_mlp(pre, ea, w1c, b1, w2, b2, w3t, b3, *, block_e):
    e, h = pre.shape
    ni = ea.shape[1]
    grid = (e // block_e,)
    return pl.pallas_call(
        _mlp_body,
        grid=grid,
        in_specs=[
            pl.BlockSpec((block_e, h), lambda i: (i, 0)),
            pl.BlockSpec((block_e, ni), lambda i: (i, 0)),
            pl.BlockSpec((ni, h), lambda i: (0, 0)),
            pl.BlockSpec((1, h), lambda i: (0, 0)),
            pl.BlockSpec((h, h), lambda i: (0, 0)),
            pl.BlockSpec((1, h), lambda i: (0, 0)),
            pl.BlockSpec((1, h), lambda i: (0, 0)),
            pl.BlockSpec((1, 1), lambda i: (0, 0)),
        ],
        out_specs=pl.BlockSpec((block_e, h), lambda i: (i, 0)),
        out_shape=jax.ShapeDtypeStruct((e, h), jnp.float32),
    )(pre, ea, w1c, b1, w2, b2, w3t, b3)


_SCH = 640
_OWN = 320
_BUF = 160
_FB = 64


def _scatter_add(wmsg, idx_r, n, *, interpret=False):
    e, h = wmsg.shape
    own = (n + _NW - 1) // _NW
    mesh = plsc.VectorSubcoreMesh(core_axis_name="c", subcore_axis_name="s",
                                  num_cores=_NC, num_subcores=_NS)

    @functools.partial(
        pl.kernel,
        out_type=jax.ShapeDtypeStruct((_NW, _OWN, h), jnp.float32),
        mesh=mesh,
        compiler_params=pltpu.CompilerParams(needs_layout_passes=False),
        scratch_types=[
            pltpu.VMEM((_SCH,), jnp.int32),
            pltpu.VMEM((_BUF,), jnp.int32),
            pltpu.VMEM((2 * _FB,), jnp.int32),
            pltpu.VMEM((2 * _FB,), jnp.int32),
            pltpu.VMEM((2 * _FB, h), jnp.float32),
            pltpu.VMEM((_OWN, h), jnp.float32),
            pltpu.SemaphoreType.DMA,
        ],
        interpret=interpret,
    )
    def k(w_hbm, ir_hbm, z_hbm, out_hbm, idxb, pb, eidb, ob, rows, acc, sem):
        c = lax.axis_index("c")
        s = lax.axis_index("s")
        wid = s * _NC + c
        lo = wid * own
        iota = lax.iota(jnp.int32, _LANES)
        zero16 = jnp.zeros((_LANES,), jnp.int32)

        for r0 in range(0, _OWN, 2 * _FB):
            step = min(2 * _FB, _OWN - r0)
            pltpu.sync_copy(z_hbm.at[pl.ds(0, step)],
                            acc.at[pl.ds(r0, step)])
        for r0 in range(0, _BUF, _LANES):
            pb[pl.ds(r0, _LANES)] = zero16

        nj = h // _LANES

        def accumulate(bs, cnt):
            def grp(q, _):
                ov = ob[pl.ds(bs + q * _LANES, _LANES)]
                dv = jnp.where(q * _LANES + iota < cnt, ov, own)
                drow = [dv[ln] for ln in range(_LANES)]
                for ln in range(_LANES):
                    d = drow[ln]
                    r = bs + q * _LANES + ln
                    sls = [pl.ds(j * _LANES, _LANES) for j in range(nj)]
                    avs = [acc[d, sl] for sl in sls]
                    rvs = [rows[r, sl] for sl in sls]
                    for j in range(nj):
                        acc[d, sls[j]] = avs[j] + rvs[j]
                return 0
            lax.fori_loop(0, _FB // _LANES, grp, 0)

        def wait_and_accumulate(bs):
            pltpu.make_async_copy(
                w_hbm.at[eidb.at[pl.ds(bs, _FB)]],
                rows.at[pl.ds(bs, _FB)], sem).wait()
            accumulate(bs, _FB)

        def issue(bs):
            for q in range(_FB // _LANES):
                pv = pb[pl.ds(q * _LANES, _LANES)]
                eidb[pl.ds(bs + q * _LANES, _LANES)] = pv >> 9
                ob[pl.ds(bs + q * _LANES, _LANES)] = pv & 0x1FF
            pltpu.async_copy(w_hbm.at[eidb.at[pl.ds(bs, _FB)]],
                             rows.at[pl.ds(bs, _FB)], sem)

        def scan_chunk(ci, carry):
            wp, fc = carry
            base = ci * _SCH
            pltpu.sync_copy(ir_hbm.at[pl.ds(base, _SCH)], idxb)

            def group(g2, carry):
                wp, fc = carry
                for u in range(2):
                    g = g2 * 2 + u
                    v = idxb[pl.ds(g * _LANES, _LANES)]
                    off = v - lo
                    m = (off >= 0) & (off < own)
                    cnt = plsc.all_reduce_population_count(m)[0]

                    @pl.when(cnt > 0)
                    def _(wp=wp, off=off, m=m, g=g):
                        packed = ((base + g * _LANES + iota) << 9) \
                            | (off & 0x1FF)
                        plsc.store_compressed(pb.at[pl.ds(wp, _LANES)],
                                              packed, mask=m)

                    wp = wp + cnt

                @pl.when(wp >= _FB)
                def _():
                    bs = (fc & 1) * _FB

                    @pl.when(fc > 0)
                    def _():
                        wait_and_accumulate(_FB - bs)

                    issue(bs)
                    pvs = [pb[pl.ds(_FB + q * _LANES, _LANES)]
                           for q in range(2)]
                    for q in range(2):
                        pb[pl.ds(q * _LANES, _LANES)] = pvs[q]

                hit = wp >= _FB
                return (jnp.where(hit, wp - _FB, wp),
                        jnp.where(hit, fc + 1, fc))

            return lax.fori_loop(0, _SCH // _LANES // 2, group, (wp, fc))

        wp, fc = lax.fori_loop(0, e // _SCH, scan_chunk,
                               (jnp.int32(0), jnp.int32(0)))

        @pl.when(fc > 0)
        def _():
            wait_and_accumulate(_FB - (fc & 1) * _FB)

        bs = (fc & 1) * _FB
        issue(bs)
        pltpu.make_async_copy(w_hbm.at[eidb.at[pl.ds(bs, _FB)]],
                              rows.at[pl.ds(bs, _FB)], sem).wait()
        accumulate(bs, wp)
        pltpu.sync_copy(acc, out_hbm.at[wid])

    out = k(wmsg, idx_r, jnp.zeros((2 * _FB, h), jnp.float32))
    return out[:, :own, :].reshape(_NW * own, h)[:n]


def _combine_body(a_ref, b_ref, o_ref):
    o_ref[...] = a_ref[...] + b_ref[...]


def _combine(a, b, *, block_n):
    n, h = a.shape
    return pl.pallas_call(
        _combine_body,
        grid=(n // block_n,),
        in_specs=[
            pl.BlockSpec((block_n, h), lambda i: (i, 0)),
            pl.BlockSpec((block_n, h), lambda i: (i, 0)),
        ],
        out_specs=pl.BlockSpec((block_n, h), lambda i: (i, 0)),
        out_shape=jax.ShapeDtypeStruct((n, h), jnp.float32),
    )(a, b)


def kernel(x, index, edge_attr, W1, b1, W2, b2, W3, b3):
    n, h = x.shape[1], x.shape[2]
    e = index.shape[1]
    w_ab = jnp.stack([W1[:h], W1[h:2 * h]])
    ps, pr = _node_proj(x, w_ab, block_n=2000)
    eh = e // 2
    outs = []
    for si in range(2):
        sl = slice(si * eh, (si + 1) * eh)
        pre = _gather_add(ps, pr, index[0, sl], index[1, sl])
        wmsg = _edge_mlp(pre, edge_attr[sl], W1[2 * h:], b1.reshape(1, h),
                         W2.astype(jnp.bfloat16), b2.reshape(1, h),
                         W3.reshape(1, h), b3.reshape(1, 1), block_e=1600)
        outs.append(_scatter_add(wmsg, index[1, sl], n))
    return _combine(outs[0], outs[1], block_n=2000)
